# Initial kernel scaffold; baseline (speedup 1.0000x reference)
import jax
import jax.numpy as jnp
from jax.experimental import pallas as pl


def kernel(x, edge_index, edge_weight, W1, b1, W2, b2, Wr, br):
    raise NotImplementedError("write your pallas kernel here")


# speedup vs baseline: 14.6104x; 14.6104x over previous
import functools

import jax
import jax.numpy as jnp
from jax import lax
from jax.experimental import pallas as pl
from jax.experimental.pallas import tpu as pltpu
from jax.experimental.pallas import tpu_sc as plsc

NC = 2
NS = 16
NW = NC * NS
CH = 2000
ZW = 10


def _mesh():
    return plsc.VectorSubcoreMesh(core_axis_name="c", subcore_axis_name="s")


_SC_PARAMS = pltpu.CompilerParams(
    needs_layout_passes=False, use_tc_tiling_on_sc=False)


def _rsqrt16(xdeg):
    yi = jnp.int32(0x5F3759DF) - (plsc.bitcast(xdeg, jnp.int32) >> 1)
    y = plsc.bitcast(yi, jnp.float32)
    for _ in range(3):
        y = y * (1.5 - 0.5 * xdeg * y * y)
    return jnp.where(xdeg > 0, y, 0.0)


def _nl1_body(N, E, src_hbm, dst_hbm, ew_hbm, xcm_hbm, zeros1_hbm,
              dinv_hbm, norm_hbm, out_hbm, tab_v, s_v, d_v, nrm_v, val_v,
              acc_sh):
    c = lax.axis_index("c")
    s = lax.axis_index("s")
    wid = c * NS + s
    epw = E // NW
    eps = E // NS
    nchunks = epw // CH
    nz = N // ZW
    zch = nz // CH

    @pl.when(s < ZW)
    def _():
        pltpu.sync_copy(zeros1_hbm, val_v)

        def zit(j, _):
            pltpu.sync_copy(val_v, acc_sh.at[pl.ds(s * nz + j * CH, CH)])
            return 0

        lax.fori_loop(0, zch, zit, 0)

    plsc.subcore_barrier()

    def dchunk(k, _):
        base = s * eps + k * CH
        pltpu.sync_copy(dst_hbm.at[pl.ds(base, CH)], d_v)
        pltpu.sync_copy(ew_hbm.at[pl.ds(base, CH)], val_v)
        pltpu.sync_copy(val_v, acc_sh.at[d_v], add=True)
        return 0

    lax.fori_loop(0, eps // CH, dchunk, 0)
    plsc.subcore_barrier()

    @pl.when(s < ZW)
    def _():
        def dv_it(j, _):
            off = s * nz + j * CH
            pltpu.sync_copy(acc_sh.at[pl.ds(off, CH)], val_v)

            def n_it(i, _):
                sl = pl.ds(i * 16, 16)
                val_v[sl] = _rsqrt16(val_v[sl] + 1.0)
                return 0

            lax.fori_loop(0, CH // 16, n_it, 0, unroll=4)
            pltpu.sync_copy(val_v, dinv_hbm.at[pl.ds(off, CH)])
            return 0

        lax.fori_loop(0, zch, dv_it, 0)

    plsc.subcore_barrier()

    for p in range(4):
        if p == 0:
            pltpu.sync_copy(dinv_hbm, tab_v)

            def chunk0(k, _):
                base = wid * epw + k * CH
                pltpu.sync_copy(src_hbm.at[pl.ds(base, CH)], s_v)
                pltpu.sync_copy(dst_hbm.at[pl.ds(base, CH)], d_v)
                pltpu.sync_copy(ew_hbm.at[pl.ds(base, CH)], nrm_v)

                def it(i, _):
                    sl = pl.ds(i * 16, 16)
                    a = plsc.load_gather(tab_v, [s_v[sl]])
                    b = plsc.load_gather(tab_v, [d_v[sl]])
                    nrm_v[sl] = a * nrm_v[sl] * b
                    return 0

                lax.fori_loop(0, CH // 16, it, 0, unroll=4)
                pltpu.sync_copy(nrm_v, norm_hbm.at[pl.ds(base, CH)])
                return 0

            lax.fori_loop(0, nchunks, chunk0, 0)
        else:
            pltpu.sync_copy(xcm_hbm.at[pl.ds((p - 1) * N, N)], tab_v)

            @pl.when(s < ZW)
            def _():
                pltpu.sync_copy(zeros1_hbm, val_v)

                def zit(j, _):
                    pltpu.sync_copy(val_v,
                                    acc_sh.at[pl.ds(s * nz + j * CH, CH)])
                    return 0

                lax.fori_loop(0, zch, zit, 0)

            plsc.subcore_barrier()

            def chunkp(k, _):
                base = wid * epw + k * CH
                pltpu.sync_copy(src_hbm.at[pl.ds(base, CH)], s_v)
                pltpu.sync_copy(dst_hbm.at[pl.ds(base, CH)], d_v)
                pltpu.sync_copy(norm_hbm.at[pl.ds(base, CH)], nrm_v)

                def it(i, _):
                    sl = pl.ds(i * 16, 16)
                    val_v[sl] = plsc.load_gather(tab_v, [s_v[sl]]) * nrm_v[sl]
                    return 0

                lax.fori_loop(0, CH // 16, it, 0, unroll=4)
                pltpu.sync_copy(val_v, acc_sh.at[d_v], add=True)
                return 0

            lax.fori_loop(0, nchunks, chunkp, 0)
            plsc.subcore_barrier()

            @pl.when(s < ZW)
            def _():
                def oit(j, _):
                    off = s * nz + j * CH
                    pltpu.sync_copy(acc_sh.at[pl.ds(off, CH)], val_v)
                    pltpu.sync_copy(
                        val_v,
                        out_hbm.at[pl.ds(c * 3 * N + (p - 1) * N + off, CH)])
                    return 0

                lax.fori_loop(0, zch, oit, 0)

            plsc.subcore_barrier()


def _nl1_call(src, dst, ew, xcm, zeros1, N, E):
    body = functools.partial(_nl1_body, N, E)
    return pl.kernel(
        body,
        out_type=(
            jax.ShapeDtypeStruct((N,), jnp.float32),
            jax.ShapeDtypeStruct((E,), jnp.float32),
            jax.ShapeDtypeStruct((NC * 3 * N,), jnp.float32),
        ),
        mesh=_mesh(),
        compiler_params=_SC_PARAMS,
        scratch_types=[
            pltpu.VMEM((N,), jnp.float32),
            pltpu.VMEM((CH,), jnp.int32),
            pltpu.VMEM((CH,), jnp.int32),
            pltpu.VMEM((CH,), jnp.float32),
            pltpu.VMEM((CH,), jnp.float32),
            pltpu.VMEM_SHARED((N,), jnp.float32),
        ],
    )(src, dst, ew, xcm, zeros1)


def _h1_body(a1p_ref, x_ref, dsq3_ref, w_ref, b_ref, o_ref):
    a1 = a1p_ref[0] + a1p_ref[1] + x_ref[...] * dsq3_ref[...]
    h1 = jnp.dot(a1, w_ref[...], preferred_element_type=jnp.float32)
    h1 = jnp.maximum(h1 + b_ref[...], 0.0)
    o_ref[0] = h1[:, :16]
    o_ref[1] = h1[:, 16:]


def _h1_call(a1p, x, dsq3, W1, b1, N, R):
    ng = N // R
    return pl.pallas_call(
        _h1_body,
        grid=(ng,),
        in_specs=[
            pl.BlockSpec((NC, R, 3), lambda i: (0, i, 0)),
            pl.BlockSpec((R, 3), lambda i: (i, 0)),
            pl.BlockSpec((R, 3), lambda i: (i, 0)),
            pl.BlockSpec((3, 32), lambda i: (0, 0)),
            pl.BlockSpec((1, 32), lambda i: (0, 0)),
        ],
        out_specs=pl.BlockSpec((NC, R, 16), lambda i: (0, i, 0)),
        out_shape=jax.ShapeDtypeStruct((NC, N, 16), jnp.float32),
    )(a1p, x, dsq3, W1, b1)


NT = 92000
RC1 = 1840


def _l2_body(N, E, src_hbm, dst_hbm, nrm_hbm, h1f_hbm, zeros16_hbm,
             a2_hbm, s_v, d_v, nrm_v, rows_v, a2_sh):
    c = lax.axis_index("c")
    s = lax.axis_index("s")
    eps = E // NS
    nchunks = eps // CH
    coff = c * N
    tail_n = N - NT
    trc = tail_n // ZW
    lanes = lax.iota(jnp.int32, 16)

    @pl.when(s < ZW)
    def _():
        pltpu.sync_copy(zeros16_hbm, rows_v)

        def zit(j, _):
            pltpu.sync_copy(rows_v.at[pl.ds(0, RC1)],
                            a2_sh.at[pl.ds(s * (NT // ZW) + j * RC1, RC1)])
            return 0

        lax.fori_loop(0, NT // ZW // RC1, zit, 0)

    plsc.subcore_barrier()

    def run_pass(keep_tail):
        def chunk(k, _):
            base = s * eps + k * CH
            pltpu.sync_copy(src_hbm.at[pl.ds(base, CH)], s_v)
            pltpu.sync_copy(dst_hbm.at[pl.ds(base, CH)], d_v)
            pltpu.sync_copy(nrm_hbm.at[pl.ds(base, CH)], nrm_v)

            def off_it(i, _):
                sl = pl.ds(i * 16, 16)
                s_v[sl] = s_v[sl] + coff
                return 0

            lax.fori_loop(0, CH // 16, off_it, 0, unroll=4)
            pltpu.sync_copy(h1f_hbm.at[s_v], rows_v)

            def fix_it(i, _):
                sl = pl.ds(i * 16, 16)
                d16 = d_v[sl]
                tail = d16 >= NT
                if keep_tail:
                    d_v[sl] = jnp.where(tail, d16 - NT, d16 & 63)
                    nrm_v[sl] = jnp.where(tail, nrm_v[sl], 0.0)
                else:
                    d_v[sl] = jnp.where(tail, d16 & 63, d16)
                    nrm_v[sl] = jnp.where(tail, 0.0, nrm_v[sl])
                return 0

            lax.fori_loop(0, CH // 16, fix_it, 0, unroll=4)

            def scale_it(j, _):
                jv = lanes * 0 + j
                nv = plsc.load_gather(nrm_v, [jv])
                v = plsc.load_gather(rows_v, [jv, lanes])
                plsc.store_scatter(rows_v, [jv, lanes], v * nv)
                return 0

            lax.fori_loop(0, CH, scale_it, 0, unroll=8)
            pltpu.sync_copy(rows_v, a2_sh.at[d_v], add=True)
            return 0

        lax.fori_loop(0, nchunks, chunk, 0)

    run_pass(False)
    plsc.subcore_barrier()

    @pl.when(s < ZW)
    def _():
        def oit(j, _):
            off = s * (NT // ZW) + j * RC1
            pltpu.sync_copy(a2_sh.at[pl.ds(off, RC1)],
                            rows_v.at[pl.ds(0, RC1)])
            pltpu.sync_copy(rows_v.at[pl.ds(0, RC1)],
                            a2_hbm.at[c, pl.ds(off, RC1)])
            return 0

        lax.fori_loop(0, NT // ZW // RC1, oit, 0)

    plsc.subcore_barrier()

    @pl.when(s < ZW)
    def _():
        pltpu.sync_copy(zeros16_hbm, rows_v)
        pltpu.sync_copy(rows_v.at[pl.ds(0, trc)],
                        a2_sh.at[pl.ds(s * trc, trc)])

    plsc.subcore_barrier()

    run_pass(True)
    plsc.subcore_barrier()

    @pl.when(s < ZW)
    def _():
        pltpu.sync_copy(a2_sh.at[pl.ds(s * trc, trc)],
                        rows_v.at[pl.ds(0, trc)])
        pltpu.sync_copy(rows_v.at[pl.ds(0, trc)],
                        a2_hbm.at[c, pl.ds(NT + s * trc, trc)])


def _l2_call(src, dst, nrm, h1f, zeros16, N, E):
    body = functools.partial(_l2_body, N, E)
    return pl.kernel(
        body,
        out_type=jax.ShapeDtypeStruct((NC, N, 16), jnp.float32),
        mesh=_mesh(),
        compiler_params=_SC_PARAMS,
        scratch_types=[
            pltpu.VMEM((CH,), jnp.int32),
            pltpu.VMEM((CH,), jnp.int32),
            pltpu.VMEM((CH,), jnp.float32),
            pltpu.VMEM((CH, 16), jnp.float32),
            pltpu.VMEM_SHARED((NT, 16), jnp.float32),
        ],
    )(src, dst, nrm, h1f, zeros16)


def _out_body(ng, ninv, a2_ref, h1_ref, dsq16_ref, w2_ref, b2_ref, wr_ref,
              br_ref, o_ref, acc):
    i = pl.program_id(0)

    @pl.when(i == 0)
    def _():
        acc[...] = jnp.zeros_like(acc)

    dsq16 = dsq16_ref[...]
    a2lo = a2_ref[0] + h1_ref[0] * dsq16
    a2hi = a2_ref[1] + h1_ref[1] * dsq16
    a2f = jnp.concatenate([a2lo, a2hi], axis=1)
    h2 = jnp.dot(a2f, w2_ref[...], preferred_element_type=jnp.float32)
    h2 = jnp.maximum(h2 + b2_ref[...], 0.0)
    acc[...] += jnp.sum(h2, axis=0, keepdims=True)

    @pl.when(i == ng - 1)
    def _():
        hg = acc[...] * ninv
        o_ref[...] = jnp.dot(hg, wr_ref[...],
                             preferred_element_type=jnp.float32) + br_ref[...]


def _out_call(a2, h1, dsq16, W2, b2, Wr, br, N, R):
    ng = N // R
    body = functools.partial(_out_body, ng, 1.0 / N)
    return pl.pallas_call(
        body,
        grid=(ng,),
        in_specs=[
            pl.BlockSpec((NC, R, 16), lambda i: (0, i, 0)),
            pl.BlockSpec((NC, R, 16), lambda i: (0, i, 0)),
            pl.BlockSpec((R, 16), lambda i: (i, 0)),
            pl.BlockSpec((32, 32), lambda i: (0, 0)),
            pl.BlockSpec((1, 32), lambda i: (0, 0)),
            pl.BlockSpec((32, 1), lambda i: (0, 0)),
            pl.BlockSpec((1, 1), lambda i: (0, 0)),
        ],
        out_specs=pl.BlockSpec((1, 1), lambda i: (0, 0)),
        out_shape=jax.ShapeDtypeStruct((1, 1), jnp.float32),
        scratch_shapes=[pltpu.VMEM((1, 32), jnp.float32)],
    )(a2, h1, dsq16, W2, b2, Wr, br)


def kernel(x, edge_index, edge_weight, W1, b1, W2, b2, Wr, br):
    N = x.shape[0]
    E = edge_weight.shape[0]
    R = 2000
    assert E % (NW * CH) == 0 and N % (ZW * 8) == 0 and N % R == 0

    f32 = jnp.float32
    src = edge_index[0]
    dst = edge_index[1]
    xcm = jnp.transpose(x).reshape(3 * N)
    zeros1 = jnp.zeros((CH,), f32)
    zeros16 = jnp.zeros((CH, 16), f32)

    dinv, norm, a1p = _nl1_call(src, dst, edge_weight, xcm, zeros1, N, E)
    dsq = dinv * dinv
    dsq3 = jnp.broadcast_to(dsq[:, None], (N, 3))
    dsq16 = jnp.broadcast_to(dsq[:, None], (N, 16))
    a1pt = jnp.transpose(a1p.reshape(NC, 3, N), (0, 2, 1))
    h1 = _h1_call(a1pt, x, dsq3, W1, b1.reshape(1, 32), N, R)
    h1f = h1.reshape(NC * N, 16)
    a2 = _l2_call(src, dst, norm, h1f, zeros16, N, E)
    res = _out_call(a2, h1, dsq16, W2, b2.reshape(1, 32), Wr,
                    br.reshape(1, 1), N, R)
    return res.reshape(1)


# speedup vs baseline: 21.6198x; 1.4798x over previous
import functools

import jax
import jax.numpy as jnp
from jax import lax
from jax.experimental import pallas as pl
from jax.experimental.pallas import tpu as pltpu
from jax.experimental.pallas import tpu_sc as plsc

NC = 2
NS = 16
NW = NC * NS
CH = 2000
ZW = 10


def _mesh():
    return plsc.VectorSubcoreMesh(core_axis_name="c", subcore_axis_name="s")


_SC_PARAMS = pltpu.CompilerParams(
    needs_layout_passes=False, use_tc_tiling_on_sc=False)


def _rsqrt16(xdeg):
    yi = jnp.int32(0x5F3759DF) - (plsc.bitcast(xdeg, jnp.int32) >> 1)
    y = plsc.bitcast(yi, jnp.float32)
    for _ in range(3):
        y = y * (1.5 - 0.5 * xdeg * y * y)
    return jnp.where(xdeg > 0, y, 0.0)


def _nl1_body(N, E, src_hbm, dst_hbm, ew_hbm, xcm_hbm, zeros1_hbm,
              dinv_hbm, norm_hbm, out_hbm, tab_v, s_v, d_v, nrm_v, val_v,
              acc_sh):
    c = lax.axis_index("c")
    s = lax.axis_index("s")
    wid = c * NS + s
    epw = E // NW
    eps = E // NS
    nchunks = epw // CH
    nz = N // ZW
    zch = nz // CH

    @pl.when(s < ZW)
    def _():
        pltpu.sync_copy(zeros1_hbm, val_v)

        def zit(j, _):
            pltpu.sync_copy(val_v, acc_sh.at[pl.ds(s * nz + j * CH, CH)])
            return 0

        lax.fori_loop(0, zch, zit, 0)

    plsc.subcore_barrier()

    def dchunk(k, _):
        base = s * eps + k * CH
        pltpu.sync_copy(dst_hbm.at[pl.ds(base, CH)], d_v)
        pltpu.sync_copy(ew_hbm.at[pl.ds(base, CH)], val_v)
        pltpu.sync_copy(val_v, acc_sh.at[d_v], add=True)
        return 0

    lax.fori_loop(0, eps // CH, dchunk, 0)
    plsc.subcore_barrier()

    @pl.when(s < ZW)
    def _():
        def dv_it(j, _):
            off = s * nz + j * CH
            pltpu.sync_copy(acc_sh.at[pl.ds(off, CH)], val_v)

            def n_it(i, _):
                sl = pl.ds(i * 16, 16)
                val_v[sl] = _rsqrt16(val_v[sl] + 1.0)
                return 0

            lax.fori_loop(0, CH // 16, n_it, 0, unroll=4)
            pltpu.sync_copy(val_v, dinv_hbm.at[pl.ds(off, CH)])
            return 0

        lax.fori_loop(0, zch, dv_it, 0)

    plsc.subcore_barrier()

    for p in range(4):
        if p == 0:
            pltpu.sync_copy(dinv_hbm, tab_v)

            def chunk0(k, _):
                base = wid * epw + k * CH
                pltpu.sync_copy(src_hbm.at[pl.ds(base, CH)], s_v)
                pltpu.sync_copy(dst_hbm.at[pl.ds(base, CH)], d_v)
                pltpu.sync_copy(ew_hbm.at[pl.ds(base, CH)], nrm_v)

                def it(i, _):
                    sl = pl.ds(i * 16, 16)
                    a = plsc.load_gather(tab_v, [s_v[sl]])
                    b = plsc.load_gather(tab_v, [d_v[sl]])
                    nrm_v[sl] = a * nrm_v[sl] * b
                    return 0

                lax.fori_loop(0, CH // 16, it, 0, unroll=4)
                pltpu.sync_copy(nrm_v, norm_hbm.at[pl.ds(base, CH)])
                return 0

            lax.fori_loop(0, nchunks, chunk0, 0)
        else:
            pltpu.sync_copy(xcm_hbm.at[pl.ds((p - 1) * N, N)], tab_v)

            @pl.when(s < ZW)
            def _():
                pltpu.sync_copy(zeros1_hbm, val_v)

                def zit(j, _):
                    pltpu.sync_copy(val_v,
                                    acc_sh.at[pl.ds(s * nz + j * CH, CH)])
                    return 0

                lax.fori_loop(0, zch, zit, 0)

            plsc.subcore_barrier()

            def chunkp(k, _):
                base = wid * epw + k * CH
                pltpu.sync_copy(src_hbm.at[pl.ds(base, CH)], s_v)
                pltpu.sync_copy(dst_hbm.at[pl.ds(base, CH)], d_v)
                pltpu.sync_copy(norm_hbm.at[pl.ds(base, CH)], nrm_v)

                def it(i, _):
                    sl = pl.ds(i * 16, 16)
                    val_v[sl] = plsc.load_gather(tab_v, [s_v[sl]]) * nrm_v[sl]
                    return 0

                lax.fori_loop(0, CH // 16, it, 0, unroll=4)
                pltpu.sync_copy(val_v, acc_sh.at[d_v], add=True)
                return 0

            lax.fori_loop(0, nchunks, chunkp, 0)
            plsc.subcore_barrier()

            @pl.when(s < ZW)
            def _():
                def oit(j, _):
                    off = s * nz + j * CH
                    pltpu.sync_copy(acc_sh.at[pl.ds(off, CH)], val_v)
                    pltpu.sync_copy(
                        val_v,
                        out_hbm.at[pl.ds(c * 3 * N + (p - 1) * N + off, CH)])
                    return 0

                lax.fori_loop(0, zch, oit, 0)

            plsc.subcore_barrier()


def _nl1_call(src, dst, ew, xcm, zeros1, N, E):
    body = functools.partial(_nl1_body, N, E)
    return pl.kernel(
        body,
        out_type=(
            jax.ShapeDtypeStruct((N,), jnp.float32),
            jax.ShapeDtypeStruct((E,), jnp.float32),
            jax.ShapeDtypeStruct((NC * 3 * N,), jnp.float32),
        ),
        mesh=_mesh(),
        compiler_params=_SC_PARAMS,
        scratch_types=[
            pltpu.VMEM((N,), jnp.float32),
            pltpu.VMEM((CH,), jnp.int32),
            pltpu.VMEM((CH,), jnp.int32),
            pltpu.VMEM((CH,), jnp.float32),
            pltpu.VMEM((CH,), jnp.float32),
            pltpu.VMEM_SHARED((N,), jnp.float32),
        ],
    )(src, dst, ew, xcm, zeros1)


def _h1_body(a1p_ref, x_ref, dsq3_ref, w_ref, b_ref, o_ref):
    a1 = a1p_ref[0] + a1p_ref[1] + x_ref[...] * dsq3_ref[...]
    h1 = jnp.dot(a1, w_ref[...], preferred_element_type=jnp.float32)
    h1 = jnp.maximum(h1 + b_ref[...], 0.0)
    o_ref[0] = h1[:, :16]
    o_ref[1] = h1[:, 16:]


def _h1_call(a1p, x, dsq3, W1, b1, N, R):
    ng = N // R
    return pl.pallas_call(
        _h1_body,
        grid=(ng,),
        in_specs=[
            pl.BlockSpec((NC, R, 3), lambda i: (0, i, 0)),
            pl.BlockSpec((R, 3), lambda i: (i, 0)),
            pl.BlockSpec((R, 3), lambda i: (i, 0)),
            pl.BlockSpec((3, 32), lambda i: (0, 0)),
            pl.BlockSpec((1, 32), lambda i: (0, 0)),
        ],
        out_specs=pl.BlockSpec((NC, R, 16), lambda i: (0, i, 0)),
        out_shape=jax.ShapeDtypeStruct((NC, N, 16), jnp.float32),
    )(a1p, x, dsq3, W1, b1)


NT = 92000
RC1 = 1840


def _l2_body(N, E, src_hbm, dst_hbm, nrm_hbm, h1f_hbm, zeros16_hbm,
             a2_hbm, s_v, d_v, nrm_v, rows_v, a2_sh):
    c = lax.axis_index("c")
    s = lax.axis_index("s")
    eps = E // NS
    nchunks = eps // CH
    coff = c * N
    tail_n = N - NT
    trc = tail_n // ZW
    lanes = lax.iota(jnp.int32, 16)

    @pl.when(s < ZW)
    def _():
        pltpu.sync_copy(zeros16_hbm, rows_v)

        def zit(j, _):
            pltpu.sync_copy(rows_v.at[pl.ds(0, RC1)],
                            a2_sh.at[pl.ds(s * (NT // ZW) + j * RC1, RC1)])
            return 0

        lax.fori_loop(0, NT // ZW // RC1, zit, 0)

    plsc.subcore_barrier()

    def run_pass(keep_tail):
        def chunk(k, _):
            base = s * eps + k * CH
            pltpu.sync_copy(src_hbm.at[pl.ds(base, CH)], s_v)
            pltpu.sync_copy(dst_hbm.at[pl.ds(base, CH)], d_v)
            pltpu.sync_copy(nrm_hbm.at[pl.ds(base, CH)], nrm_v)

            def off_it(i, _):
                sl = pl.ds(i * 16, 16)
                s_v[sl] = s_v[sl] + coff
                return 0

            lax.fori_loop(0, CH // 16, off_it, 0, unroll=4)
            pltpu.sync_copy(h1f_hbm.at[s_v], rows_v)

            def fix_it(i, _):
                sl = pl.ds(i * 16, 16)
                d16 = d_v[sl]
                tail = d16 >= NT
                if keep_tail:
                    d_v[sl] = jnp.where(tail, d16 - NT, d16 & 63)
                    nrm_v[sl] = jnp.where(tail, nrm_v[sl], 0.0)
                else:
                    d_v[sl] = jnp.where(tail, d16 & 63, d16)
                    nrm_v[sl] = jnp.where(tail, 0.0, nrm_v[sl])
                return 0

            lax.fori_loop(0, CH // 16, fix_it, 0, unroll=4)

            def scale_blk(i, _):
                nrm16 = nrm_v[pl.ds(i * 16, 16)]
                for k in range(16):
                    r = i * 16 + k
                    rows_v[r] = rows_v[r] * nrm16[k]
                return 0

            lax.fori_loop(0, CH // 16, scale_blk, 0)
            pltpu.sync_copy(rows_v, a2_sh.at[d_v], add=True)
            return 0

        lax.fori_loop(0, nchunks, chunk, 0)

    run_pass(False)
    plsc.subcore_barrier()

    @pl.when(s < ZW)
    def _():
        def oit(j, _):
            off = s * (NT // ZW) + j * RC1
            pltpu.sync_copy(a2_sh.at[pl.ds(off, RC1)],
                            rows_v.at[pl.ds(0, RC1)])
            pltpu.sync_copy(rows_v.at[pl.ds(0, RC1)],
                            a2_hbm.at[c, pl.ds(off, RC1)])
            return 0

        lax.fori_loop(0, NT // ZW // RC1, oit, 0)

    plsc.subcore_barrier()

    @pl.when(s < ZW)
    def _():
        pltpu.sync_copy(zeros16_hbm, rows_v)
        pltpu.sync_copy(rows_v.at[pl.ds(0, trc)],
                        a2_sh.at[pl.ds(s * trc, trc)])

    plsc.subcore_barrier()

    run_pass(True)
    plsc.subcore_barrier()

    @pl.when(s < ZW)
    def _():
        pltpu.sync_copy(a2_sh.at[pl.ds(s * trc, trc)],
                        rows_v.at[pl.ds(0, trc)])
        pltpu.sync_copy(rows_v.at[pl.ds(0, trc)],
                        a2_hbm.at[c, pl.ds(NT + s * trc, trc)])


def _l2_call(src, dst, nrm, h1f, zeros16, N, E):
    body = functools.partial(_l2_body, N, E)
    return pl.kernel(
        body,
        out_type=jax.ShapeDtypeStruct((NC, N, 16), jnp.float32),
        mesh=_mesh(),
        compiler_params=_SC_PARAMS,
        scratch_types=[
            pltpu.VMEM((CH,), jnp.int32),
            pltpu.VMEM((CH,), jnp.int32),
            pltpu.VMEM((CH,), jnp.float32),
            pltpu.VMEM((CH, 16), jnp.float32),
            pltpu.VMEM_SHARED((NT, 16), jnp.float32),
        ],
    )(src, dst, nrm, h1f, zeros16)


def _out_body(ng, ninv, a2_ref, h1_ref, dsq16_ref, w2_ref, b2_ref, wr_ref,
              br_ref, o_ref, acc):
    i = pl.program_id(0)

    @pl.when(i == 0)
    def _():
        acc[...] = jnp.zeros_like(acc)

    dsq16 = dsq16_ref[...]
    a2lo = a2_ref[0] + h1_ref[0] * dsq16
    a2hi = a2_ref[1] + h1_ref[1] * dsq16
    a2f = jnp.concatenate([a2lo, a2hi], axis=1)
    h2 = jnp.dot(a2f, w2_ref[...], preferred_element_type=jnp.float32)
    h2 = jnp.maximum(h2 + b2_ref[...], 0.0)
    acc[...] += jnp.sum(h2, axis=0, keepdims=True)

    @pl.when(i == ng - 1)
    def _():
        hg = acc[...] * ninv
        o_ref[...] = jnp.dot(hg, wr_ref[...],
                             preferred_element_type=jnp.float32) + br_ref[...]


def _out_call(a2, h1, dsq16, W2, b2, Wr, br, N, R):
    ng = N // R
    body = functools.partial(_out_body, ng, 1.0 / N)
    return pl.pallas_call(
        body,
        grid=(ng,),
        in_specs=[
            pl.BlockSpec((NC, R, 16), lambda i: (0, i, 0)),
            pl.BlockSpec((NC, R, 16), lambda i: (0, i, 0)),
            pl.BlockSpec((R, 16), lambda i: (i, 0)),
            pl.BlockSpec((32, 32), lambda i: (0, 0)),
            pl.BlockSpec((1, 32), lambda i: (0, 0)),
            pl.BlockSpec((32, 1), lambda i: (0, 0)),
            pl.BlockSpec((1, 1), lambda i: (0, 0)),
        ],
        out_specs=pl.BlockSpec((1, 1), lambda i: (0, 0)),
        out_shape=jax.ShapeDtypeStruct((1, 1), jnp.float32),
        scratch_shapes=[pltpu.VMEM((1, 32), jnp.float32)],
    )(a2, h1, dsq16, W2, b2, Wr, br)


def kernel(x, edge_index, edge_weight, W1, b1, W2, b2, Wr, br):
    N = x.shape[0]
    E = edge_weight.shape[0]
    R = 2000
    assert E % (NW * CH) == 0 and N % (ZW * 8) == 0 and N % R == 0

    f32 = jnp.float32
    src = edge_index[0]
    dst = edge_index[1]
    xcm = jnp.transpose(x).reshape(3 * N)
    zeros1 = jnp.zeros((CH,), f32)
    zeros16 = jnp.zeros((CH, 16), f32)

    dinv, norm, a1p = _nl1_call(src, dst, edge_weight, xcm, zeros1, N, E)
    dsq = dinv * dinv
    dsq3 = jnp.broadcast_to(dsq[:, None], (N, 3))
    dsq16 = jnp.broadcast_to(dsq[:, None], (N, 16))
    a1pt = jnp.transpose(a1p.reshape(NC, 3, N), (0, 2, 1))
    h1 = _h1_call(a1pt, x, dsq3, W1, b1.reshape(1, 32), N, R)
    h1f = h1.reshape(NC * N, 16)
    a2 = _l2_call(src, dst, norm, h1f, zeros16, N, E)
    res = _out_call(a2, h1, dsq16, W2, b2.reshape(1, 32), Wr,
                    br.reshape(1, 1), N, R)
    return res.reshape(1)


# speedup vs baseline: 23.3182x; 1.0786x over previous
import functools

import jax
import jax.numpy as jnp
from jax import lax
from jax.experimental import pallas as pl
from jax.experimental.pallas import tpu as pltpu
from jax.experimental.pallas import tpu_sc as plsc

NC = 2
NS = 16
NW = NC * NS
CH = 1000
ZW = 10


def _mesh():
    return plsc.VectorSubcoreMesh(core_axis_name="c", subcore_axis_name="s")


_SC_PARAMS = pltpu.CompilerParams(
    needs_layout_passes=False, use_tc_tiling_on_sc=False)


def _rsqrt16(xdeg):
    yi = jnp.int32(0x5F3759DF) - (plsc.bitcast(xdeg, jnp.int32) >> 1)
    y = plsc.bitcast(yi, jnp.float32)
    for _ in range(3):
        y = y * (1.5 - 0.5 * xdeg * y * y)
    return jnp.where(xdeg > 0, y, 0.0)


def _nl1_body(N, E, src_hbm, dst_hbm, ew_hbm, xcm_hbm, zeros1_hbm,
              dinv_hbm, norm_hbm, out_hbm, tab_v, s_v, d_v, nrm_v, val_v,
              acc_sh):
    c = lax.axis_index("c")
    s = lax.axis_index("s")
    wid = c * NS + s
    epw = E // NW
    eps = E // NS
    nchunks = epw // CH
    nz = N // ZW
    zch = nz // CH

    @pl.when(s < ZW)
    def _():
        pltpu.sync_copy(zeros1_hbm, val_v)

        def zit(j, _):
            pltpu.sync_copy(val_v, acc_sh.at[pl.ds(s * nz + j * CH, CH)])
            return 0

        lax.fori_loop(0, zch, zit, 0)

    plsc.subcore_barrier()

    def dchunk(k, _):
        base = s * eps + k * CH
        pltpu.sync_copy(dst_hbm.at[pl.ds(base, CH)], d_v)
        pltpu.sync_copy(ew_hbm.at[pl.ds(base, CH)], val_v)
        pltpu.sync_copy(val_v, acc_sh.at[d_v], add=True)
        return 0

    lax.fori_loop(0, eps // CH, dchunk, 0)
    plsc.subcore_barrier()

    @pl.when(s < ZW)
    def _():
        def dv_it(j, _):
            off = s * nz + j * CH
            pltpu.sync_copy(acc_sh.at[pl.ds(off, CH)], val_v)

            def n_it(i, _):
                sl = pl.ds(i * 16, 16)
                val_v[sl] = _rsqrt16(val_v[sl] + 1.0)
                return 0

            lax.fori_loop(0, CH // 16, n_it, 0, unroll=4)
            pltpu.sync_copy(val_v, dinv_hbm.at[pl.ds(off, CH)])
            return 0

        lax.fori_loop(0, zch, dv_it, 0)

    plsc.subcore_barrier()

    for p in range(4):
        if p == 0:
            pltpu.sync_copy(dinv_hbm, tab_v)

            def chunk0(k, _):
                base = wid * epw + k * CH
                pltpu.sync_copy(src_hbm.at[pl.ds(base, CH)], s_v)
                pltpu.sync_copy(dst_hbm.at[pl.ds(base, CH)], d_v)
                pltpu.sync_copy(ew_hbm.at[pl.ds(base, CH)], nrm_v)

                def it(i, _):
                    sl = pl.ds(i * 16, 16)
                    a = plsc.load_gather(tab_v, [s_v[sl]])
                    b = plsc.load_gather(tab_v, [d_v[sl]])
                    nrm_v[sl] = a * nrm_v[sl] * b
                    return 0

                lax.fori_loop(0, CH // 16, it, 0, unroll=4)
                pltpu.sync_copy(nrm_v, norm_hbm.at[pl.ds(base, CH)])
                return 0

            lax.fori_loop(0, nchunks, chunk0, 0)
        else:
            pltpu.sync_copy(xcm_hbm.at[pl.ds((p - 1) * N, N)], tab_v)

            @pl.when(s < ZW)
            def _():
                pltpu.sync_copy(zeros1_hbm, val_v)

                def zit(j, _):
                    pltpu.sync_copy(val_v,
                                    acc_sh.at[pl.ds(s * nz + j * CH, CH)])
                    return 0

                lax.fori_loop(0, zch, zit, 0)

            plsc.subcore_barrier()

            def chunkp(k, _):
                base = wid * epw + k * CH
                pltpu.sync_copy(src_hbm.at[pl.ds(base, CH)], s_v)
                pltpu.sync_copy(dst_hbm.at[pl.ds(base, CH)], d_v)
                pltpu.sync_copy(norm_hbm.at[pl.ds(base, CH)], nrm_v)

                def it(i, _):
                    sl = pl.ds(i * 16, 16)
                    val_v[sl] = plsc.load_gather(tab_v, [s_v[sl]]) * nrm_v[sl]
                    return 0

                lax.fori_loop(0, CH // 16, it, 0, unroll=4)
                pltpu.sync_copy(val_v, acc_sh.at[d_v], add=True)
                return 0

            lax.fori_loop(0, nchunks, chunkp, 0)
            plsc.subcore_barrier()

            @pl.when(s < ZW)
            def _():
                def oit(j, _):
                    off = s * nz + j * CH
                    pltpu.sync_copy(acc_sh.at[pl.ds(off, CH)], val_v)
                    pltpu.sync_copy(
                        val_v,
                        out_hbm.at[pl.ds(c * 3 * N + (p - 1) * N + off, CH)])
                    return 0

                lax.fori_loop(0, zch, oit, 0)

            plsc.subcore_barrier()


def _nl1_call(src, dst, ew, xcm, zeros1, N, E):
    body = functools.partial(_nl1_body, N, E)
    return pl.kernel(
        body,
        out_type=(
            jax.ShapeDtypeStruct((N,), jnp.float32),
            jax.ShapeDtypeStruct((E,), jnp.float32),
            jax.ShapeDtypeStruct((NC * 3 * N,), jnp.float32),
        ),
        mesh=_mesh(),
        compiler_params=_SC_PARAMS,
        scratch_types=[
            pltpu.VMEM((N,), jnp.float32),
            pltpu.VMEM((CH,), jnp.int32),
            pltpu.VMEM((CH,), jnp.int32),
            pltpu.VMEM((CH,), jnp.float32),
            pltpu.VMEM((CH,), jnp.float32),
            pltpu.VMEM_SHARED((N,), jnp.float32),
        ],
    )(src, dst, ew, xcm, zeros1)


def _h1_body(a1p_ref, x_ref, dsq3_ref, w_ref, b_ref, o_ref):
    a1 = a1p_ref[0] + a1p_ref[1] + x_ref[...] * dsq3_ref[...]
    h1 = jnp.dot(a1, w_ref[...], preferred_element_type=jnp.float32)
    h1 = jnp.maximum(h1 + b_ref[...], 0.0)
    o_ref[0] = h1[:, :16]
    o_ref[1] = h1[:, 16:]


def _h1_call(a1p, x, dsq3, W1, b1, N, R):
    ng = N // R
    return pl.pallas_call(
        _h1_body,
        grid=(ng,),
        in_specs=[
            pl.BlockSpec((NC, R, 3), lambda i: (0, i, 0)),
            pl.BlockSpec((R, 3), lambda i: (i, 0)),
            pl.BlockSpec((R, 3), lambda i: (i, 0)),
            pl.BlockSpec((3, 32), lambda i: (0, 0)),
            pl.BlockSpec((1, 32), lambda i: (0, 0)),
        ],
        out_specs=pl.BlockSpec((NC, R, 16), lambda i: (0, i, 0)),
        out_shape=jax.ShapeDtypeStruct((NC, N, 16), jnp.float32),
    )(a1p, x, dsq3, W1, b1)


NT = 100000
RC1 = 1000


def _l2_body(N, E, src_hbm, dst_hbm, nrm_hbm, h1f_hbm, zeros16_hbm,
             a2_hbm, s_v, d_v, nrm_v, rows_v, a2_sh):
    c = lax.axis_index("c")
    s = lax.axis_index("s")
    eps = E // NS
    nchunks = eps // CH
    coff = c * N
    tail_n = N - NT
    trc = tail_n // ZW
    lanes = lax.iota(jnp.int32, 16)

    @pl.when(s < ZW)
    def _():
        pltpu.sync_copy(zeros16_hbm, rows_v)

        def zit(j, _):
            pltpu.sync_copy(rows_v.at[pl.ds(0, RC1)],
                            a2_sh.at[pl.ds(s * (NT // ZW) + j * RC1, RC1)])
            return 0

        lax.fori_loop(0, NT // ZW // RC1, zit, 0)

    plsc.subcore_barrier()

    def run_pass(keep_tail):
        def chunk(k, _):
            base = s * eps + k * CH
            pltpu.sync_copy(src_hbm.at[pl.ds(base, CH)], s_v)
            pltpu.sync_copy(dst_hbm.at[pl.ds(base, CH)], d_v)
            pltpu.sync_copy(nrm_hbm.at[pl.ds(base, CH)], nrm_v)

            def off_it(i, _):
                sl = pl.ds(i * 16, 16)
                s_v[sl] = s_v[sl] + coff
                return 0

            lax.fori_loop(0, CH // 16, off_it, 0, unroll=4)
            pltpu.sync_copy(h1f_hbm.at[s_v], rows_v)

            def fix_it(i, _):
                sl = pl.ds(i * 16, 16)
                d16 = d_v[sl]
                tail = d16 >= NT
                if keep_tail:
                    d_v[sl] = jnp.where(tail, d16 - NT, d16 & 63)
                    nrm_v[sl] = jnp.where(tail, nrm_v[sl], 0.0)
                else:
                    d_v[sl] = jnp.where(tail, d16 & 63, d16)
                    nrm_v[sl] = jnp.where(tail, 0.0, nrm_v[sl])
                return 0

            lax.fori_loop(0, CH // 16, fix_it, 0, unroll=4)

            def scale_blk(i, _):
                nrm16 = nrm_v[pl.ds(i * 16, 16)]
                for k in range(16):
                    r = i * 16 + k
                    rows_v[r] = rows_v[r] * nrm16[k]
                return 0

            lax.fori_loop(0, CH // 16, scale_blk, 0)
            pltpu.sync_copy(rows_v, a2_sh.at[d_v], add=True)
            return 0

        lax.fori_loop(0, nchunks, chunk, 0)

    run_pass(False)
    plsc.subcore_barrier()

    @pl.when(s < ZW)
    def _():
        def oit(j, _):
            off = s * (NT // ZW) + j * RC1
            pltpu.sync_copy(a2_sh.at[pl.ds(off, RC1)],
                            rows_v.at[pl.ds(0, RC1)])
            pltpu.sync_copy(rows_v.at[pl.ds(0, RC1)],
                            a2_hbm.at[c, pl.ds(off, RC1)])
            return 0

        lax.fori_loop(0, NT // ZW // RC1, oit, 0)

    plsc.subcore_barrier()

    if tail_n > 0:
        @pl.when(s < ZW)
        def _():
            pltpu.sync_copy(zeros16_hbm, rows_v)
            pltpu.sync_copy(rows_v.at[pl.ds(0, trc)],
                            a2_sh.at[pl.ds(s * trc, trc)])

        plsc.subcore_barrier()

        run_pass(True)
        plsc.subcore_barrier()

        @pl.when(s < ZW)
        def _():
            pltpu.sync_copy(a2_sh.at[pl.ds(s * trc, trc)],
                            rows_v.at[pl.ds(0, trc)])
            pltpu.sync_copy(rows_v.at[pl.ds(0, trc)],
                            a2_hbm.at[c, pl.ds(NT + s * trc, trc)])


def _l2_call(src, dst, nrm, h1f, zeros16, N, E):
    body = functools.partial(_l2_body, N, E)
    return pl.kernel(
        body,
        out_type=jax.ShapeDtypeStruct((NC, N, 16), jnp.float32),
        mesh=_mesh(),
        compiler_params=_SC_PARAMS,
        scratch_types=[
            pltpu.VMEM((CH,), jnp.int32),
            pltpu.VMEM((CH,), jnp.int32),
            pltpu.VMEM((CH,), jnp.float32),
            pltpu.VMEM((CH, 16), jnp.float32),
            pltpu.VMEM_SHARED((NT, 16), jnp.float32),
        ],
    )(src, dst, nrm, h1f, zeros16)


def _out_body(ng, ninv, a2_ref, h1_ref, dsq16_ref, w2_ref, b2_ref, wr_ref,
              br_ref, o_ref, acc):
    i = pl.program_id(0)

    @pl.when(i == 0)
    def _():
        acc[...] = jnp.zeros_like(acc)

    dsq16 = dsq16_ref[...]
    a2lo = a2_ref[0] + h1_ref[0] * dsq16
    a2hi = a2_ref[1] + h1_ref[1] * dsq16
    a2f = jnp.concatenate([a2lo, a2hi], axis=1)
    h2 = jnp.dot(a2f, w2_ref[...], preferred_element_type=jnp.float32)
    h2 = jnp.maximum(h2 + b2_ref[...], 0.0)
    acc[...] += jnp.sum(h2, axis=0, keepdims=True)

    @pl.when(i == ng - 1)
    def _():
        hg = acc[...] * ninv
        o_ref[...] = jnp.dot(hg, wr_ref[...],
                             preferred_element_type=jnp.float32) + br_ref[...]


def _out_call(a2, h1, dsq16, W2, b2, Wr, br, N, R):
    ng = N // R
    body = functools.partial(_out_body, ng, 1.0 / N)
    return pl.pallas_call(
        body,
        grid=(ng,),
        in_specs=[
            pl.BlockSpec((NC, R, 16), lambda i: (0, i, 0)),
            pl.BlockSpec((NC, R, 16), lambda i: (0, i, 0)),
            pl.BlockSpec((R, 16), lambda i: (i, 0)),
            pl.BlockSpec((32, 32), lambda i: (0, 0)),
            pl.BlockSpec((1, 32), lambda i: (0, 0)),
            pl.BlockSpec((32, 1), lambda i: (0, 0)),
            pl.BlockSpec((1, 1), lambda i: (0, 0)),
        ],
        out_specs=pl.BlockSpec((1, 1), lambda i: (0, 0)),
        out_shape=jax.ShapeDtypeStruct((1, 1), jnp.float32),
        scratch_shapes=[pltpu.VMEM((1, 32), jnp.float32)],
    )(a2, h1, dsq16, W2, b2, Wr, br)


def kernel(x, edge_index, edge_weight, W1, b1, W2, b2, Wr, br):
    N = x.shape[0]
    E = edge_weight.shape[0]
    R = 2000
    assert E % (NW * CH) == 0 and N % (ZW * 8) == 0 and N % R == 0

    f32 = jnp.float32
    src = edge_index[0]
    dst = edge_index[1]
    xcm = jnp.transpose(x).reshape(3 * N)
    zeros1 = jnp.zeros((CH,), f32)
    zeros16 = jnp.zeros((CH, 16), f32)

    dinv, norm, a1p = _nl1_call(src, dst, edge_weight, xcm, zeros1, N, E)
    dsq = dinv * dinv
    dsq3 = jnp.broadcast_to(dsq[:, None], (N, 3))
    dsq16 = jnp.broadcast_to(dsq[:, None], (N, 16))
    a1pt = jnp.transpose(a1p.reshape(NC, 3, N), (0, 2, 1))
    h1 = _h1_call(a1pt, x, dsq3, W1, b1.reshape(1, 32), N, R)
    h1f = h1.reshape(NC * N, 16)
    a2 = _l2_call(src, dst, norm, h1f, zeros16, N, E)
    res = _out_call(a2, h1, dsq16, W2, b2.reshape(1, 32), Wr,
                    br.reshape(1, 1), N, R)
    return res.reshape(1)


# speedup vs baseline: 27.5629x; 1.1820x over previous
import functools

import jax
import jax.numpy as jnp
from jax import lax
from jax.experimental import pallas as pl
from jax.experimental.pallas import tpu as pltpu
from jax.experimental.pallas import tpu_sc as plsc

NC = 2
NS = 16
NW = NC * NS
CH = 2000
ZW = 10


def _mesh():
    return plsc.VectorSubcoreMesh(core_axis_name="c", subcore_axis_name="s")


_SC_PARAMS = pltpu.CompilerParams(
    needs_layout_passes=False, use_tc_tiling_on_sc=False)


def _rsqrt16(xdeg):
    yi = jnp.int32(0x5F3759DF) - (plsc.bitcast(xdeg, jnp.int32) >> 1)
    y = plsc.bitcast(yi, jnp.float32)
    for _ in range(3):
        y = y * (1.5 - 0.5 * xdeg * y * y)
    return jnp.where(xdeg > 0, y, 0.0)


def _nl1_body(N, E, src_hbm, dst_hbm, ew_hbm, xcm_hbm, zeros1_hbm,
              dinv_hbm, norm_hbm, out_hbm, tab_v, s_v, d_v, nrm_v, val_v,
              acc_sh):
    c = lax.axis_index("c")
    s = lax.axis_index("s")
    wid = c * NS + s
    epw = E // NW
    eps = E // NS
    nchunks = epw // CH
    nz = N // ZW
    zch = nz // CH

    @pl.when(s < ZW)
    def _():
        pltpu.sync_copy(zeros1_hbm, val_v)

        def zit(j, _):
            pltpu.sync_copy(val_v, acc_sh.at[pl.ds(s * nz + j * CH, CH)])
            return 0

        lax.fori_loop(0, zch, zit, 0)

    plsc.subcore_barrier()

    def dchunk(k, _):
        base = s * eps + k * CH
        pltpu.sync_copy(dst_hbm.at[pl.ds(base, CH)], d_v)
        pltpu.sync_copy(ew_hbm.at[pl.ds(base, CH)], val_v)
        pltpu.sync_copy(val_v, acc_sh.at[d_v], add=True)
        return 0

    lax.fori_loop(0, eps // CH, dchunk, 0)
    plsc.subcore_barrier()

    @pl.when(s < ZW)
    def _():
        def dv_it(j, _):
            off = s * nz + j * CH
            pltpu.sync_copy(acc_sh.at[pl.ds(off, CH)], val_v)

            def n_it(i, _):
                sl = pl.ds(i * 16, 16)
                val_v[sl] = _rsqrt16(val_v[sl] + 1.0)
                return 0

            lax.fori_loop(0, CH // 16, n_it, 0, unroll=4)
            pltpu.sync_copy(val_v, dinv_hbm.at[pl.ds(off, CH)])
            return 0

        lax.fori_loop(0, zch, dv_it, 0)

    plsc.subcore_barrier()

    for p in range(4):
        if p == 0:
            pltpu.sync_copy(dinv_hbm, tab_v)

            def chunk0(k, _):
                base = wid * epw + k * CH
                pltpu.sync_copy(src_hbm.at[pl.ds(base, CH)], s_v)
                pltpu.sync_copy(dst_hbm.at[pl.ds(base, CH)], d_v)
                pltpu.sync_copy(ew_hbm.at[pl.ds(base, CH)], nrm_v)

                def it(i, _):
                    sl = pl.ds(i * 16, 16)
                    a = plsc.load_gather(tab_v, [s_v[sl]])
                    b = plsc.load_gather(tab_v, [d_v[sl]])
                    nrm_v[sl] = a * nrm_v[sl] * b
                    return 0

                lax.fori_loop(0, CH // 16, it, 0, unroll=4)
                pltpu.sync_copy(nrm_v, norm_hbm.at[pl.ds(base, CH)])
                return 0

            lax.fori_loop(0, nchunks, chunk0, 0)
        else:
            pltpu.sync_copy(xcm_hbm.at[pl.ds((p - 1) * N, N)], tab_v)

            @pl.when(s < ZW)
            def _():
                pltpu.sync_copy(zeros1_hbm, val_v)

                def zit(j, _):
                    pltpu.sync_copy(val_v,
                                    acc_sh.at[pl.ds(s * nz + j * CH, CH)])
                    return 0

                lax.fori_loop(0, zch, zit, 0)

            plsc.subcore_barrier()

            def chunkp(k, _):
                base = wid * epw + k * CH
                pltpu.sync_copy(src_hbm.at[pl.ds(base, CH)], s_v)
                pltpu.sync_copy(dst_hbm.at[pl.ds(base, CH)], d_v)
                pltpu.sync_copy(norm_hbm.at[pl.ds(base, CH)], nrm_v)

                def it(i, _):
                    sl = pl.ds(i * 16, 16)
                    val_v[sl] = plsc.load_gather(tab_v, [s_v[sl]]) * nrm_v[sl]
                    return 0

                lax.fori_loop(0, CH // 16, it, 0, unroll=4)
                pltpu.sync_copy(val_v, acc_sh.at[d_v], add=True)
                return 0

            lax.fori_loop(0, nchunks, chunkp, 0)
            plsc.subcore_barrier()

            @pl.when(s < ZW)
            def _():
                def oit(j, _):
                    off = s * nz + j * CH
                    pltpu.sync_copy(acc_sh.at[pl.ds(off, CH)], val_v)
                    pltpu.sync_copy(
                        val_v,
                        out_hbm.at[pl.ds(c * 3 * N + (p - 1) * N + off, CH)])
                    return 0

                lax.fori_loop(0, zch, oit, 0)

            plsc.subcore_barrier()


def _nl1_call(src, dst, ew, xcm, zeros1, N, E):
    body = functools.partial(_nl1_body, N, E)
    return pl.kernel(
        body,
        out_type=(
            jax.ShapeDtypeStruct((N,), jnp.float32),
            jax.ShapeDtypeStruct((E,), jnp.float32),
            jax.ShapeDtypeStruct((NC * 3 * N,), jnp.float32),
        ),
        mesh=_mesh(),
        compiler_params=_SC_PARAMS,
        scratch_types=[
            pltpu.VMEM((N,), jnp.float32),
            pltpu.VMEM((CH,), jnp.int32),
            pltpu.VMEM((CH,), jnp.int32),
            pltpu.VMEM((CH,), jnp.float32),
            pltpu.VMEM((CH,), jnp.float32),
            pltpu.VMEM_SHARED((N,), jnp.float32),
        ],
    )(src, dst, ew, xcm, zeros1)


def _h1_body(a1p_ref, x_ref, dsq3_ref, w_ref, b_ref, o_ref):
    a1 = a1p_ref[0] + a1p_ref[1] + x_ref[...] * dsq3_ref[...]
    h1 = jnp.dot(a1, w_ref[...], preferred_element_type=jnp.float32)
    h1 = jnp.maximum(h1 + b_ref[...], 0.0)
    o_ref[0] = h1[:, :16]
    o_ref[1] = h1[:, 16:]


def _h1_call(a1p, x, dsq3, W1, b1, N, R):
    ng = N // R
    return pl.pallas_call(
        _h1_body,
        grid=(ng,),
        in_specs=[
            pl.BlockSpec((NC, R, 3), lambda i: (0, i, 0)),
            pl.BlockSpec((R, 3), lambda i: (i, 0)),
            pl.BlockSpec((R, 3), lambda i: (i, 0)),
            pl.BlockSpec((3, 32), lambda i: (0, 0)),
            pl.BlockSpec((1, 32), lambda i: (0, 0)),
        ],
        out_specs=pl.BlockSpec((NC, R, 16), lambda i: (0, i, 0)),
        out_shape=jax.ShapeDtypeStruct((NC, N, 16), jnp.float32),
    )(a1p, x, dsq3, W1, b1)


def _l2_body(N, E, src_hbm, dst_hbm, nrm_hbm, h1f_hbm, zeros16_hbm,
             a2_hbm, s1_v, s2_v, d1_v, d2_v, nrm_v, rows_v, a2_sh):
    c = lax.axis_index("c")
    s = lax.axis_index("s")
    eps = E // NS
    nchunks = eps // CH
    coff = c * N
    hh = CH // 2
    rc = 1000
    nz = N // ZW

    @pl.when(s < ZW)
    def _():
        pltpu.sync_copy(zeros16_hbm, rows_v.at[pl.ds(0, rc)])

        def zit(j, _):
            pltpu.sync_copy(rows_v.at[pl.ds(0, rc)],
                            a2_sh.at[pl.ds(s * nz + j * rc, rc)])
            return 0

        lax.fori_loop(0, nz // rc, zit, 0)

    plsc.subcore_barrier()

    h0, h1 = 1008, CH - 1008

    def chunk(k, _):
        base = s * eps + k * CH
        pltpu.sync_copy(src_hbm.at[pl.ds(base, h0)], s1_v)
        pltpu.sync_copy(src_hbm.at[pl.ds(base + h0, h1)], s2_v)
        pltpu.sync_copy(dst_hbm.at[pl.ds(base, h0)], d1_v)
        pltpu.sync_copy(dst_hbm.at[pl.ds(base + h0, h1)], d2_v)
        pltpu.sync_copy(nrm_hbm.at[pl.ds(base, CH)], nrm_v)

        def off1_it(i, _):
            sl = pl.ds(i * 16, 16)
            s1_v[sl] = s1_v[sl] + coff
            return 0

        def off2_it(i, _):
            sl = pl.ds(i * 16, 16)
            s2_v[sl] = s2_v[sl] + coff
            return 0

        lax.fori_loop(0, h0 // 16, off1_it, 0, unroll=4)
        lax.fori_loop(0, h1 // 16, off2_it, 0, unroll=4)

        for noff, hlen, sv, dv in ((0, h0, s1_v, d1_v),
                                   (h0, h1, s2_v, d2_v)):
            if hlen == h0:
                pltpu.sync_copy(h1f_hbm.at[sv], rows_v)
            else:
                pltpu.sync_copy(h1f_hbm.at[sv], rows_v.at[pl.ds(0, h1)])

            def scale_blk(i, _, noff=noff):
                nrm16 = nrm_v[pl.ds(noff + i * 16, 16)]
                for kk in range(16):
                    r = i * 16 + kk
                    rows_v[r] = rows_v[r] * nrm16[kk]
                return 0

            lax.fori_loop(0, hlen // 16, scale_blk, 0)
            if hlen == h0:
                pltpu.sync_copy(rows_v, a2_sh.at[dv], add=True)
            else:
                pltpu.sync_copy(rows_v.at[pl.ds(0, h1)],
                                a2_sh.at[dv], add=True)
        return 0

    lax.fori_loop(0, nchunks, chunk, 0)
    plsc.subcore_barrier()

    @pl.when(s < ZW)
    def _():
        def oit(j, _):
            off = s * nz + j * rc
            pltpu.sync_copy(a2_sh.at[pl.ds(off, rc)], rows_v.at[pl.ds(0, rc)])
            pltpu.sync_copy(rows_v.at[pl.ds(0, rc)],
                            a2_hbm.at[c, pl.ds(off, rc)])
            return 0

        lax.fori_loop(0, nz // rc, oit, 0)


def _l2_call(src, dst, nrm, h1f, zeros16, N, E):
    body = functools.partial(_l2_body, N, E)
    return pl.kernel(
        body,
        out_type=jax.ShapeDtypeStruct((NC, N, 16), jnp.float32),
        mesh=_mesh(),
        compiler_params=_SC_PARAMS,
        scratch_types=[
            pltpu.VMEM((1008,), jnp.int32),
            pltpu.VMEM((992,), jnp.int32),
            pltpu.VMEM((1008,), jnp.int32),
            pltpu.VMEM((992,), jnp.int32),
            pltpu.VMEM((CH,), jnp.float32),
            pltpu.VMEM((1008, 16), jnp.float32),
            pltpu.VMEM_SHARED((N, 16), jnp.float32),
        ],
    )(src, dst, nrm, h1f, zeros16)


def _out_body(ng, ninv, a2_ref, h1_ref, dsq16_ref, w2_ref, b2_ref, wr_ref,
              br_ref, o_ref, acc):
    i = pl.program_id(0)

    @pl.when(i == 0)
    def _():
        acc[...] = jnp.zeros_like(acc)

    dsq16 = dsq16_ref[...]
    a2lo = a2_ref[0] + h1_ref[0] * dsq16
    a2hi = a2_ref[1] + h1_ref[1] * dsq16
    a2f = jnp.concatenate([a2lo, a2hi], axis=1)
    h2 = jnp.dot(a2f, w2_ref[...], preferred_element_type=jnp.float32)
    h2 = jnp.maximum(h2 + b2_ref[...], 0.0)
    acc[...] += jnp.sum(h2, axis=0, keepdims=True)

    @pl.when(i == ng - 1)
    def _():
        hg = acc[...] * ninv
        o_ref[...] = jnp.dot(hg, wr_ref[...],
                             preferred_element_type=jnp.float32) + br_ref[...]


def _out_call(a2, h1, dsq16, W2, b2, Wr, br, N, R):
    ng = N // R
    body = functools.partial(_out_body, ng, 1.0 / N)
    return pl.pallas_call(
        body,
        grid=(ng,),
        in_specs=[
            pl.BlockSpec((NC, R, 16), lambda i: (0, i, 0)),
            pl.BlockSpec((NC, R, 16), lambda i: (0, i, 0)),
            pl.BlockSpec((R, 16), lambda i: (i, 0)),
            pl.BlockSpec((32, 32), lambda i: (0, 0)),
            pl.BlockSpec((1, 32), lambda i: (0, 0)),
            pl.BlockSpec((32, 1), lambda i: (0, 0)),
            pl.BlockSpec((1, 1), lambda i: (0, 0)),
        ],
        out_specs=pl.BlockSpec((1, 1), lambda i: (0, 0)),
        out_shape=jax.ShapeDtypeStruct((1, 1), jnp.float32),
        scratch_shapes=[pltpu.VMEM((1, 32), jnp.float32)],
    )(a2, h1, dsq16, W2, b2, Wr, br)


def kernel(x, edge_index, edge_weight, W1, b1, W2, b2, Wr, br):
    N = x.shape[0]
    E = edge_weight.shape[0]
    R = 2000
    assert E % (NW * CH) == 0 and N % (ZW * 8) == 0 and N % R == 0

    f32 = jnp.float32
    src = edge_index[0]
    dst = edge_index[1]
    xcm = jnp.transpose(x).reshape(3 * N)
    zeros1 = jnp.zeros((CH,), f32)
    zeros16 = jnp.zeros((1000, 16), f32)

    dinv, norm, a1p = _nl1_call(src, dst, edge_weight, xcm, zeros1, N, E)
    dsq = dinv * dinv
    dsq3 = jnp.broadcast_to(dsq[:, None], (N, 3))
    dsq16 = jnp.broadcast_to(dsq[:, None], (N, 16))
    a1pt = jnp.transpose(a1p.reshape(NC, 3, N), (0, 2, 1))
    h1 = _h1_call(a1pt, x, dsq3, W1, b1.reshape(1, 32), N, R)
    h1f = h1.reshape(NC * N, 16)
    a2 = _l2_call(src, dst, norm, h1f, zeros16, N, E)
    res = _out_call(a2, h1, dsq16, W2, b2.reshape(1, 32), Wr,
                    br.reshape(1, 1), N, R)
    return res.reshape(1)
